# baseline (speedup 1.0000x reference)
import jax
import jax.numpy as jnp
from jax.experimental import pallas as pl
from jax.experimental.pallas import tpu as pltpu

B = 4096
IN_DIMS = 1024
HID = 1024
OUT = 1024
CH = 1024
NCH = B // CH
NBUF = 3


def _body(x_hbm, ws_hbm, bs_ref, g0_ref, b0_ref, we_hbm, be_ref, out_hbm,
          xbuf, obuf, h_ref, wsf_ref, wef_ref, wsb_ref, web_ref,
          in_sems, w_sems, out_sems):
    in_cps = [
        pltpu.make_async_copy(x_hbm.at[pl.ds(c * CH, CH), :],
                              xbuf.at[c % NBUF], in_sems.at[c % NBUF])
        for c in range(NCH)
    ]
    ws_cp = pltpu.make_async_copy(ws_hbm, wsf_ref, w_sems.at[0])
    we_cp = pltpu.make_async_copy(we_hbm, wef_ref, w_sems.at[1])
    ws_cp.start()
    for c in range(NBUF):
        in_cps[c].start()
    we_cp.start()
    ws_cp.wait()
    wsb_ref[...] = wsf_ref[...].astype(jnp.bfloat16)

    ps = jnp.zeros((1, HID), jnp.float32)
    pq = jnp.zeros((1, HID), jnp.float32)
    for c in range(NCH):
        in_cps[c].wait()
        xb = xbuf[c % NBUF].astype(jnp.bfloat16)
        h = jnp.dot(xb, wsb_ref[...], preferred_element_type=jnp.float32)
        h = jnp.maximum(h + bs_ref[...], 0.0)
        h_ref[pl.ds(c * CH, CH), :] = h.astype(jnp.bfloat16)
        ps = ps + jnp.sum(h, axis=0, keepdims=True)
        pq = pq + jnp.sum(h * h, axis=0, keepdims=True)
        if c + NBUF < NCH:
            in_cps[c + NBUF].start()

    we_cp.wait()
    web_ref[...] = wef_ref[...].astype(jnp.bfloat16)
    m = ps / B
    v = pq / B - m * m
    s = g0_ref[...] * jax.lax.rsqrt(v + 1e-5)
    sb = s.astype(jnp.bfloat16)
    t = (b0_ref[...] - m * s).astype(jnp.bfloat16)
    crow = jnp.dot(t, web_ref[...], preferred_element_type=jnp.float32)
    crow = crow + be_ref[...]

    out_cps = [
        pltpu.make_async_copy(obuf.at[c % 2],
                              out_hbm.at[pl.ds(c * CH, CH), :],
                              out_sems.at[c % 2])
        for c in range(NCH)
    ]
    for c in range(NCH):
        hn = h_ref[pl.ds(c * CH, CH), :] * sb
        o = jnp.dot(hn, web_ref[...], preferred_element_type=jnp.float32)
        if c >= 2:
            out_cps[c - 2].wait()
        obuf[c % 2] = jnp.maximum(o + crow, 0.0)
        out_cps[c].start()
    out_cps[NCH - 2].wait()
    out_cps[NCH - 1].wait()


def kernel(x, W_start, b_start, bn0_g, bn0_b, W_exp, b_exp, bn_g, bn_b,
           W_end, b_end, W_dqn, b_dqn):
    del W_exp, b_exp, bn_g, bn_b, W_dqn, b_dqn
    bs = b_start.reshape(1, HID)
    g0 = bn0_g.reshape(1, HID)
    b0 = bn0_b.reshape(1, HID)
    be = b_end.reshape(1, OUT)
    any_spec = pl.BlockSpec(memory_space=pltpu.MemorySpace.HBM)
    vmem_spec = pl.BlockSpec(memory_space=pltpu.MemorySpace.VMEM)
    return pl.pallas_call(
        _body,
        in_specs=[
            any_spec,
            any_spec,
            vmem_spec,
            vmem_spec,
            vmem_spec,
            any_spec,
            vmem_spec,
        ],
        out_specs=any_spec,
        out_shape=jax.ShapeDtypeStruct((B, OUT), jnp.float32),
        scratch_shapes=[
            pltpu.VMEM((NBUF, CH, IN_DIMS), jnp.float32),
            pltpu.VMEM((2, CH, OUT), jnp.float32),
            pltpu.VMEM((B, HID), jnp.bfloat16),
            pltpu.VMEM((IN_DIMS, HID), jnp.float32),
            pltpu.VMEM((HID, OUT), jnp.float32),
            pltpu.VMEM((IN_DIMS, HID), jnp.bfloat16),
            pltpu.VMEM((HID, OUT), jnp.bfloat16),
            pltpu.SemaphoreType.DMA((NBUF,)),
            pltpu.SemaphoreType.DMA((2,)),
            pltpu.SemaphoreType.DMA((2,)),
        ],
    )(x, W_start, bs, g0, b0, W_end, be)


# speedup vs baseline: 1.0182x; 1.0182x over previous
import jax
import jax.numpy as jnp
from jax.experimental import pallas as pl
from jax.experimental.pallas import tpu as pltpu

B = 4096
IN_DIMS = 1024
HID = 1024
OUT = 1024
CH = 1024
NCH = B // CH
NBUF = 2


def _body(x_hbm, ws_hbm, bs_ref, g0_ref, b0_ref, we_hbm, be_ref, out_hbm,
          xbuf, obuf, h_ref, wsf_ref, wef_ref, wsb_ref, web_ref,
          in_sems, w_sems, out_sems):
    in_cps = [
        pltpu.make_async_copy(x_hbm.at[pl.ds(c * CH, CH), :],
                              xbuf.at[c % NBUF], in_sems.at[c % NBUF])
        for c in range(NCH)
    ]
    ws_cp = pltpu.make_async_copy(ws_hbm, wsf_ref, w_sems.at[0])
    we_cp = pltpu.make_async_copy(we_hbm, wef_ref, w_sems.at[1])
    ws_cp.start()
    for c in range(NBUF):
        in_cps[c].start()
    we_cp.start()
    ws_cp.wait()
    wsb_ref[...] = wsf_ref[...].astype(jnp.bfloat16)

    ps = jnp.zeros((1, HID), jnp.float32)
    pq = jnp.zeros((1, HID), jnp.float32)
    for c in range(NCH):
        in_cps[c].wait()
        xb = xbuf[c % NBUF].astype(jnp.bfloat16)
        h = jnp.dot(xb, wsb_ref[...], preferred_element_type=jnp.float32)
        h = jnp.maximum(h + bs_ref[...], 0.0)
        h_ref[pl.ds(c * CH, CH), :] = h.astype(jnp.bfloat16)
        ps = ps + jnp.sum(h, axis=0, keepdims=True)
        pq = pq + jnp.sum(h * h, axis=0, keepdims=True)
        if c + NBUF < NCH:
            in_cps[c + NBUF].start()

    we_cp.wait()
    web_ref[...] = wef_ref[...].astype(jnp.bfloat16)
    m = ps / B
    v = pq / B - m * m
    s = g0_ref[...] * jax.lax.rsqrt(v + 1e-5)
    sb = s.astype(jnp.bfloat16)
    t = (b0_ref[...] - m * s).astype(jnp.bfloat16)
    crow = jnp.dot(t, web_ref[...], preferred_element_type=jnp.float32)
    crow = crow + be_ref[...]

    out_cps = [
        pltpu.make_async_copy(obuf.at[c % 2],
                              out_hbm.at[pl.ds(c * CH, CH), :],
                              out_sems.at[c % 2])
        for c in range(NCH)
    ]
    for c in range(NCH):
        hn = h_ref[pl.ds(c * CH, CH), :] * sb
        o = jnp.dot(hn, web_ref[...], preferred_element_type=jnp.float32)
        if c >= 2:
            out_cps[c - 2].wait()
        obuf[c % 2] = jnp.maximum(o + crow, 0.0)
        out_cps[c].start()
    out_cps[NCH - 2].wait()
    out_cps[NCH - 1].wait()


def kernel(x, W_start, b_start, bn0_g, bn0_b, W_exp, b_exp, bn_g, bn_b,
           W_end, b_end, W_dqn, b_dqn):
    del W_exp, b_exp, bn_g, bn_b, W_dqn, b_dqn
    bs = b_start.reshape(1, HID)
    g0 = bn0_g.reshape(1, HID)
    b0 = bn0_b.reshape(1, HID)
    be = b_end.reshape(1, OUT)
    any_spec = pl.BlockSpec(memory_space=pltpu.MemorySpace.HBM)
    vmem_spec = pl.BlockSpec(memory_space=pltpu.MemorySpace.VMEM)
    return pl.pallas_call(
        _body,
        in_specs=[
            any_spec,
            any_spec,
            vmem_spec,
            vmem_spec,
            vmem_spec,
            any_spec,
            vmem_spec,
        ],
        out_specs=any_spec,
        out_shape=jax.ShapeDtypeStruct((B, OUT), jnp.float32),
        scratch_shapes=[
            pltpu.VMEM((NBUF, CH, IN_DIMS), jnp.float32),
            pltpu.VMEM((2, CH, OUT), jnp.float32),
            pltpu.VMEM((B, HID), jnp.bfloat16),
            pltpu.VMEM((IN_DIMS, HID), jnp.float32),
            pltpu.VMEM((HID, OUT), jnp.float32),
            pltpu.VMEM((IN_DIMS, HID), jnp.bfloat16),
            pltpu.VMEM((HID, OUT), jnp.bfloat16),
            pltpu.SemaphoreType.DMA((NBUF,)),
            pltpu.SemaphoreType.DMA((2,)),
            pltpu.SemaphoreType.DMA((2,)),
        ],
    )(x, W_start, bs, g0, b0, W_end, be)
